# speedup vs baseline: 1.0476x; 1.0476x over previous
import functools

import jax
import jax.numpy as jnp
from jax.experimental import pallas as pl

N = 10000
E = 320000
H = 128
L = 2

_BLK = 1000


def _matmul_body(h_ref, w_ref, o_ref):
    o_ref[...] = jnp.dot(h_ref[...], w_ref[...],
                         preferred_element_type=jnp.float32)


def _tc_matmul(h, w):
    return pl.pallas_call(
        _matmul_body,
        grid=(N // _BLK,),
        in_specs=[
            pl.BlockSpec((_BLK, H), lambda i: (i, 0)),
            pl.BlockSpec((H, H), lambda i: (0, 0)),
        ],
        out_specs=pl.BlockSpec((_BLK, H), lambda i: (i, 0)),
        out_shape=jax.ShapeDtypeStruct((N, H), jnp.float32),
    )(h, w)


def _gru_body(a_ref, h_ref, wih_ref, whh_ref, bih_ref, bhh_ref, o_ref):
    a = a_ref[...]
    h = h_ref[...]
    gi = jnp.dot(a, wih_ref[...], preferred_element_type=jnp.float32) + bih_ref[...]
    gh = jnp.dot(h, whh_ref[...], preferred_element_type=jnp.float32) + bhh_ref[...]
    r = jax.nn.sigmoid(gi[:, :H] + gh[:, :H])
    z = jax.nn.sigmoid(gi[:, H:2 * H] + gh[:, H:2 * H])
    n = jnp.tanh(gi[:, 2 * H:] + r * gh[:, 2 * H:])
    o_ref[...] = (1.0 - z) * n + z * h


def _tc_gru(agg, h, w_ih_t, w_hh_t, b_ih, b_hh):
    return pl.pallas_call(
        _gru_body,
        grid=(N // _BLK,),
        in_specs=[
            pl.BlockSpec((_BLK, H), lambda i: (i, 0)),
            pl.BlockSpec((_BLK, H), lambda i: (i, 0)),
            pl.BlockSpec((H, 3 * H), lambda i: (0, 0)),
            pl.BlockSpec((H, 3 * H), lambda i: (0, 0)),
            pl.BlockSpec((1, 3 * H), lambda i: (0, 0)),
            pl.BlockSpec((1, 3 * H), lambda i: (0, 0)),
        ],
        out_specs=pl.BlockSpec((_BLK, H), lambda i: (i, 0)),
        out_shape=jax.ShapeDtypeStruct((N, H), jnp.float32),
    )(agg, h, w_ih_t, w_hh_t, b_ih, b_hh)


def kernel(x, edge_index, edge_attr, weight, w_ih, w_hh, b_ih, b_hh):
    src = edge_index[0]
    dst = edge_index[1]
    w_ih_t = w_ih.T
    w_hh_t = w_hh.T
    b_ih2 = b_ih.reshape(1, 3 * H)
    b_hh2 = b_hh.reshape(1, 3 * H)
    h = x
    for i in range(L):
        m = _tc_matmul(h, weight[i])
        msg = m[src] * edge_attr[:, None]
        agg = jax.ops.segment_sum(msg, dst, num_segments=N)
        h = _tc_gru(agg, h, w_ih_t, w_hh_t, b_ih2, b_hh2)
    return h


# speedup vs baseline: 3.1338x; 2.9915x over previous
import dataclasses
import functools

import jax
import jax.numpy as jnp
from jax import lax
from jax.experimental import pallas as pl
from jax.experimental.pallas import tpu as pltpu
from jax.experimental.pallas import tpu_sc as plsc

N = 10000
E = 320000
H = 128
L = 2

NC = 2
NS = 16
NW = NC * NS
C = 128
EPW = -(-E // NW)
NCH = -(-EPW // C)
EPWP = NCH * C
NP = 10240
ROWS_PER_SUB = NP // NS

_BLK = 1000


def _matmul_body(h_ref, w_ref, o_ref):
    o_ref[...] = jnp.dot(h_ref[...], w_ref[...],
                         preferred_element_type=jnp.float32)


def _tc_matmul(h, w):
    return pl.pallas_call(
        _matmul_body,
        grid=(N // _BLK,),
        in_specs=[
            pl.BlockSpec((_BLK, H), lambda i: (i, 0)),
            pl.BlockSpec((H, H), lambda i: (0, 0)),
        ],
        out_specs=pl.BlockSpec((_BLK, H), lambda i: (i, 0)),
        out_shape=jax.ShapeDtypeStruct((N, H), jnp.float32),
    )(h, w)


def _gru_body(a0_ref, a1_ref, h_ref, wih_ref, whh_ref, bih_ref, bhh_ref,
              o_ref):
    a = a0_ref[0] + a1_ref[0]
    h = h_ref[...]
    gi = jnp.dot(a, wih_ref[...], preferred_element_type=jnp.float32) + bih_ref[...]
    gh = jnp.dot(h, whh_ref[...], preferred_element_type=jnp.float32) + bhh_ref[...]
    r = jax.nn.sigmoid(gi[:, :H] + gh[:, :H])
    z = jax.nn.sigmoid(gi[:, H:2 * H] + gh[:, H:2 * H])
    n = jnp.tanh(gi[:, 2 * H:] + r * gh[:, 2 * H:])
    o_ref[...] = (1.0 - z) * n + z * h


def _tc_gru(agg2, h, w_ih_t, w_hh_t, b_ih, b_hh):
    return pl.pallas_call(
        _gru_body,
        grid=(N // _BLK,),
        in_specs=[
            pl.BlockSpec((1, _BLK, H), lambda i: (0, i, 0)),
            pl.BlockSpec((1, _BLK, H), lambda i: (1, i, 0)),
            pl.BlockSpec((_BLK, H), lambda i: (i, 0)),
            pl.BlockSpec((H, 3 * H), lambda i: (0, 0)),
            pl.BlockSpec((H, 3 * H), lambda i: (0, 0)),
            pl.BlockSpec((1, 3 * H), lambda i: (0, 0)),
            pl.BlockSpec((1, 3 * H), lambda i: (0, 0)),
        ],
        out_specs=pl.BlockSpec((_BLK, H), lambda i: (i, 0)),
        out_shape=jax.ShapeDtypeStruct((N, H), jnp.float32),
    )(agg2, agg2, h, w_ih_t, w_hh_t, b_ih, b_hh)


@functools.cache
def _sc_agg_kernel():
    mesh = plsc.VectorSubcoreMesh(core_axis_name="c", subcore_axis_name="s")
    cp = pltpu.CompilerParams()
    if "needs_layout_passes" in pltpu.CompilerParams.__dataclass_fields__:
        cp = dataclasses.replace(cp, needs_layout_passes=False)
    return pl.kernel(
        _sc_agg_body,
        compiler_params=cp,
        out_type=jax.ShapeDtypeStruct((NC, NP, H), jnp.float32),
        mesh=mesh,
        scratch_types=[
            pltpu.VMEM((C,), jnp.int32),
            pltpu.VMEM((C,), jnp.int32),
            pltpu.VMEM((C,), jnp.float32),
            pltpu.VMEM((C, H), jnp.float32),
            pltpu.VMEM_SHARED((NP, H), jnp.float32),
        ],
    )


def _sc_agg_body(m_hbm, src_hbm, dst_hbm, attr_hbm, zero_hbm, out_hbm,
                 src_v, dst_v, attr_v, rows_v, agg_sh):
    c = lax.axis_index("c")
    s = lax.axis_index("s")
    w = c * NS + s

    pltpu.sync_copy(zero_hbm, agg_sh.at[pl.ds(s * ROWS_PER_SUB, ROWS_PER_SUB)])
    plsc.subcore_barrier()

    @pl.loop(0, NCH)
    def _(j):
        pltpu.sync_copy(src_hbm.at[w, j], src_v)
        pltpu.sync_copy(dst_hbm.at[w, j], dst_v)
        pltpu.sync_copy(attr_hbm.at[w, j], attr_v)
        pltpu.sync_copy(m_hbm.at[src_v], rows_v)

        @pl.loop(0, C)
        def _(e):
            splat = plsc.load_gather(attr_v, [jnp.full((16,), e, jnp.int32)])
            for k in range(H // 16):
                sl = pl.ds(k * 16, 16)
                rows_v[e, sl] = rows_v[e, sl] * splat

        pltpu.sync_copy(rows_v, agg_sh.at[dst_v], add=True)

    plsc.subcore_barrier()
    pltpu.sync_copy(agg_sh.at[pl.ds(s * ROWS_PER_SUB, ROWS_PER_SUB)],
                    out_hbm.at[c, pl.ds(s * ROWS_PER_SUB, ROWS_PER_SUB)])


def _prep_edges(edge_index, edge_attr):
    src = edge_index[0].reshape(NW, EPW)
    dst = edge_index[1].reshape(NW, EPW)
    attr = edge_attr.reshape(NW, EPW)
    pad = EPWP - EPW
    src = jnp.pad(src, ((0, 0), (0, pad))).reshape(NW, NCH, C)
    dst = jnp.pad(dst, ((0, 0), (0, pad))).reshape(NW, NCH, C)
    attr = jnp.pad(attr, ((0, 0), (0, pad))).reshape(NW, NCH, C)
    return src, dst, attr


def kernel(x, edge_index, edge_attr, weight, w_ih, w_hh, b_ih, b_hh):
    src, dst, attr = _prep_edges(edge_index, edge_attr)
    zero = jnp.zeros((ROWS_PER_SUB, H), jnp.float32)
    w_ih_t = w_ih.T
    w_hh_t = w_hh.T
    b_ih2 = b_ih.reshape(1, 3 * H)
    b_hh2 = b_hh.reshape(1, 3 * H)
    h = x
    for i in range(L):
        m = _tc_matmul(h, weight[i])
        agg2 = _sc_agg_kernel()(m, src, dst, attr, zero)
        h = _tc_gru(agg2, h, w_ih_t, w_hh_t, b_ih2, b_hh2)
    return h


# speedup vs baseline: 3.6711x; 1.1715x over previous
import dataclasses
import functools

import jax
import jax.numpy as jnp
from jax import lax
from jax.experimental import pallas as pl
from jax.experimental.pallas import tpu as pltpu
from jax.experimental.pallas import tpu_sc as plsc

N = 10000
E = 320000
H = 128
L = 2

NC = 2
NS = 16
NW = NC * NS
C = 128
RB = 4
EPW = -(-E // NW)
NCH = 2 * (-(-EPW // (2 * C)))
EPWP = NCH * C
NP = 10240
ROWS_PER_SUB = NP // NS

_BLK = 1000


def _matmul_body(h_ref, w_ref, o_ref):
    o_ref[...] = jnp.dot(h_ref[...], w_ref[...],
                         preferred_element_type=jnp.float32)


def _tc_matmul(h, w):
    return pl.pallas_call(
        _matmul_body,
        grid=(N // _BLK,),
        in_specs=[
            pl.BlockSpec((_BLK, H), lambda i: (i, 0)),
            pl.BlockSpec((H, H), lambda i: (0, 0)),
        ],
        out_specs=pl.BlockSpec((_BLK, H), lambda i: (i, 0)),
        out_shape=jax.ShapeDtypeStruct((N, H), jnp.float32),
    )(h, w)


def _gru_body(a0_ref, a1_ref, h_ref, wih_ref, whh_ref, bih_ref, bhh_ref,
              o_ref):
    a = a0_ref[0] + a1_ref[0]
    h = h_ref[...]
    gi = jnp.dot(a, wih_ref[...], preferred_element_type=jnp.float32) + bih_ref[...]
    gh = jnp.dot(h, whh_ref[...], preferred_element_type=jnp.float32) + bhh_ref[...]
    r = jax.nn.sigmoid(gi[:, :H] + gh[:, :H])
    z = jax.nn.sigmoid(gi[:, H:2 * H] + gh[:, H:2 * H])
    n = jnp.tanh(gi[:, 2 * H:] + r * gh[:, 2 * H:])
    o_ref[...] = (1.0 - z) * n + z * h


def _tc_gru(agg2, h, w_ih_t, w_hh_t, b_ih, b_hh):
    return pl.pallas_call(
        _gru_body,
        grid=(N // _BLK,),
        in_specs=[
            pl.BlockSpec((1, _BLK, H), lambda i: (0, i, 0)),
            pl.BlockSpec((1, _BLK, H), lambda i: (1, i, 0)),
            pl.BlockSpec((_BLK, H), lambda i: (i, 0)),
            pl.BlockSpec((H, 3 * H), lambda i: (0, 0)),
            pl.BlockSpec((H, 3 * H), lambda i: (0, 0)),
            pl.BlockSpec((1, 3 * H), lambda i: (0, 0)),
            pl.BlockSpec((1, 3 * H), lambda i: (0, 0)),
        ],
        out_specs=pl.BlockSpec((_BLK, H), lambda i: (i, 0)),
        out_shape=jax.ShapeDtypeStruct((N, H), jnp.float32),
    )(agg2, agg2, h, w_ih_t, w_hh_t, b_ih, b_hh)


@functools.cache
def _sc_agg_kernel():
    mesh = plsc.VectorSubcoreMesh(core_axis_name="c", subcore_axis_name="s")
    cp = pltpu.CompilerParams()
    if "needs_layout_passes" in pltpu.CompilerParams.__dataclass_fields__:
        cp = dataclasses.replace(cp, needs_layout_passes=False)
    return pl.kernel(
        _sc_agg_body,
        compiler_params=cp,
        out_type=jax.ShapeDtypeStruct((NC, NP, H), jnp.float32),
        mesh=mesh,
        scratch_types=[
            pltpu.VMEM((RB, 3, C), jnp.int32),
            pltpu.VMEM((C, H), jnp.float32),
            pltpu.VMEM((C, H), jnp.float32),
            pltpu.SemaphoreType.DMA,
            pltpu.SemaphoreType.DMA,
            pltpu.SemaphoreType.DMA,
            pltpu.SemaphoreType.DMA,
            pltpu.VMEM_SHARED((NP, H), jnp.float32),
        ],
    )


def _scale_rows(rows_v, attr_ref):
    @plsc.parallel_loop(0, C, unroll=4)
    def _(e):
        splat_i = plsc.load_gather(attr_ref, [jnp.full((16,), e, jnp.int32)])
        splat = plsc.bitcast(splat_i, jnp.float32)
        for k in range(H // 16):
            sl = pl.ds(k * 16, 16)
            rows_v[e, sl] = rows_v[e, sl] * splat


def _sc_agg_body(m_hbm, meta_hbm, zero_hbm, out_hbm,
                 meta_v, rows0, rows1, sem0, sem1, semm0, semm1, agg_sh):
    c = lax.axis_index("c")
    s = lax.axis_index("s")
    w = c * NS + s

    pltpu.sync_copy(meta_hbm.at[w, 0], meta_v.at[0])
    pltpu.sync_copy(meta_hbm.at[w, 1], meta_v.at[1])
    pltpu.async_copy(meta_hbm.at[w, 2], meta_v.at[2], semm0)
    pltpu.async_copy(meta_hbm.at[w, 3], meta_v.at[3], semm1)
    pltpu.sync_copy(zero_hbm, agg_sh.at[pl.ds(s * ROWS_PER_SUB, ROWS_PER_SUB)])
    plsc.subcore_barrier()
    pltpu.async_copy(m_hbm.at[meta_v.at[0, 0]], rows0, sem0)
    pltpu.async_copy(m_hbm.at[meta_v.at[1, 0]], rows1, sem1)

    @pl.loop(0, NCH, step=2)
    def _(j):
        def _half(jj, rows_v, sem, semm):
            slot = lax.rem(jj, RB)
            pltpu.make_async_copy(m_hbm.at[meta_v.at[slot, 0]], rows_v,
                                  sem).wait()
            _scale_rows(rows_v, meta_v.at[slot, 2])
            pltpu.sync_copy(rows_v, agg_sh.at[meta_v.at[slot, 1]], add=True)

            @pl.when(jj + 2 < NCH)
            def _():
                nslot = lax.rem(jj + 2, RB)
                pltpu.make_async_copy(meta_hbm.at[w, jj + 2],
                                      meta_v.at[nslot], semm).wait()
                pltpu.async_copy(m_hbm.at[meta_v.at[nslot, 0]], rows_v, sem)

            @pl.when(jj + 4 < NCH)
            def _():
                pltpu.async_copy(meta_hbm.at[w, jj + 4], meta_v.at[slot], semm)

        _half(j, rows0, sem0, semm0)
        _half(j + 1, rows1, sem1, semm1)

    plsc.subcore_barrier()
    pltpu.sync_copy(agg_sh.at[pl.ds(s * ROWS_PER_SUB, ROWS_PER_SUB)],
                    out_hbm.at[c, pl.ds(s * ROWS_PER_SUB, ROWS_PER_SUB)])


def _prep_edges(edge_index, edge_attr):
    src = edge_index[0].reshape(NW, EPW)
    dst = edge_index[1].reshape(NW, EPW)
    attr = lax.bitcast_convert_type(edge_attr, jnp.int32).reshape(NW, EPW)
    pad = EPWP - EPW
    src = jnp.pad(src, ((0, 0), (0, pad))).reshape(NW, NCH, C)
    dst = jnp.pad(dst, ((0, 0), (0, pad))).reshape(NW, NCH, C)
    attr = jnp.pad(attr, ((0, 0), (0, pad))).reshape(NW, NCH, C)
    return jnp.stack([src, dst, attr], axis=2)


def kernel(x, edge_index, edge_attr, weight, w_ih, w_hh, b_ih, b_hh):
    meta = _prep_edges(edge_index, edge_attr)
    zero = jnp.zeros((ROWS_PER_SUB, H), jnp.float32)
    w_ih_t = w_ih.T
    w_hh_t = w_hh.T
    b_ih2 = b_ih.reshape(1, 3 * H)
    b_hh2 = b_hh.reshape(1, 3 * H)
    h = x
    for i in range(L):
        m = _tc_matmul(h, weight[i])
        agg2 = _sc_agg_kernel()(m, meta, zero)
        h = _tc_gru(agg2, h, w_ih_t, w_hh_t, b_ih2, b_hh2)
    return h


# speedup vs baseline: 6.4905x; 1.7680x over previous
import dataclasses
import functools

import jax
import jax.numpy as jnp
from jax import lax
from jax.experimental import pallas as pl
from jax.experimental.pallas import tpu as pltpu
from jax.experimental.pallas import tpu_sc as plsc

N = 10000
E = 320000
H = 128
L = 2

NC = 2
NS = 16
NW = NC * NS
C = 120
NB = 3
RB = 6
EPW = -(-E // NW)
NCH = NB * (-(-EPW // (NB * C)))
EPWP = NCH * C
NP = 10112
ROWS_PER_SUB = NP // NS

_BLK = 1000


def _matmul_body(h_ref, w_ref, o_ref):
    o_ref[...] = jnp.dot(h_ref[...], w_ref[...],
                         preferred_element_type=jnp.float32)


def _tc_matmul(h, w):
    return pl.pallas_call(
        _matmul_body,
        grid=(N // _BLK,),
        in_specs=[
            pl.BlockSpec((_BLK, H), lambda i: (i, 0)),
            pl.BlockSpec((H, H), lambda i: (0, 0)),
        ],
        out_specs=pl.BlockSpec((_BLK, H), lambda i: (i, 0)),
        out_shape=jax.ShapeDtypeStruct((N, H), jnp.float32),
    )(h, w)


def _gru_body(a0_ref, a1_ref, h_ref, wih_ref, whh_ref, bih_ref, bhh_ref,
              o_ref):
    a = a0_ref[0] + a1_ref[0]
    h = h_ref[...]
    gi = jnp.dot(a, wih_ref[...], preferred_element_type=jnp.float32) + bih_ref[...]
    gh = jnp.dot(h, whh_ref[...], preferred_element_type=jnp.float32) + bhh_ref[...]
    r = jax.nn.sigmoid(gi[:, :H] + gh[:, :H])
    z = jax.nn.sigmoid(gi[:, H:2 * H] + gh[:, H:2 * H])
    n = jnp.tanh(gi[:, 2 * H:] + r * gh[:, 2 * H:])
    o_ref[...] = (1.0 - z) * n + z * h


def _tc_gru(agg2, h, w_ih_t, w_hh_t, b_ih, b_hh):
    return pl.pallas_call(
        _gru_body,
        grid=(N // _BLK,),
        in_specs=[
            pl.BlockSpec((1, _BLK, H), lambda i: (0, i, 0)),
            pl.BlockSpec((1, _BLK, H), lambda i: (1, i, 0)),
            pl.BlockSpec((_BLK, H), lambda i: (i, 0)),
            pl.BlockSpec((H, 3 * H), lambda i: (0, 0)),
            pl.BlockSpec((H, 3 * H), lambda i: (0, 0)),
            pl.BlockSpec((1, 3 * H), lambda i: (0, 0)),
            pl.BlockSpec((1, 3 * H), lambda i: (0, 0)),
        ],
        out_specs=pl.BlockSpec((_BLK, H), lambda i: (i, 0)),
        out_shape=jax.ShapeDtypeStruct((N, H), jnp.float32),
    )(agg2, agg2, h, w_ih_t, w_hh_t, b_ih, b_hh)


@functools.cache
def _sc_agg_kernel():
    mesh = plsc.VectorSubcoreMesh(core_axis_name="c", subcore_axis_name="s")
    cp = pltpu.CompilerParams()
    if "needs_layout_passes" in pltpu.CompilerParams.__dataclass_fields__:
        cp = dataclasses.replace(cp, needs_layout_passes=False)
    return pl.kernel(
        _sc_agg_body,
        compiler_params=cp,
        out_type=jax.ShapeDtypeStruct((NC, NP, H), jnp.float32),
        mesh=mesh,
        scratch_types=[
            pltpu.VMEM((RB, 3, C), jnp.int32),
            pltpu.VMEM((C, H), jnp.float32),
            pltpu.VMEM((C, H), jnp.float32),
            pltpu.VMEM((C, H), jnp.float32),
            pltpu.SemaphoreType.DMA,
            pltpu.SemaphoreType.DMA,
            pltpu.SemaphoreType.DMA,
            pltpu.SemaphoreType.DMA,
            pltpu.SemaphoreType.DMA,
            pltpu.SemaphoreType.DMA,
            pltpu.SemaphoreType.DMA,
            pltpu.SemaphoreType.DMA,
            pltpu.SemaphoreType.DMA,
            pltpu.VMEM_SHARED((NP, H), jnp.float32),
        ],
    )


def _scale_rows(rows_v, attr_ref):
    @plsc.parallel_loop(0, C, unroll=4)
    def _(e):
        splat_i = plsc.load_gather(attr_ref, [jnp.full((16,), e, jnp.int32)])
        splat = plsc.bitcast(splat_i, jnp.float32)
        for k in range(H // 16):
            sl = pl.ds(k * 16, 16)
            rows_v[e, sl] = rows_v[e, sl] * splat


def _sc_agg_body(m_hbm, meta_hbm, zero_hbm, out_hbm,
                 meta_v, rows0, rows1, rows2,
                 semg0, semg1, semg2, sems0, sems1, sems2,
                 semm0, semm1, semm2, agg_sh):
    c = lax.axis_index("c")
    s = lax.axis_index("s")
    w = c * NS + s
    rows = (rows0, rows1, rows2)
    semg = (semg0, semg1, semg2)
    sems = (sems0, sems1, sems2)
    semm = (semm0, semm1, semm2)

    pltpu.sync_copy(meta_hbm.at[w, 0], meta_v.at[0])
    pltpu.sync_copy(meta_hbm.at[w, 1], meta_v.at[1])
    pltpu.async_copy(meta_hbm.at[w, 2], meta_v.at[2], semm[2])
    pltpu.async_copy(meta_hbm.at[w, 3], meta_v.at[3], semm[0])
    pltpu.sync_copy(zero_hbm, agg_sh.at[pl.ds(s * ROWS_PER_SUB, ROWS_PER_SUB)])
    plsc.subcore_barrier()
    pltpu.async_copy(m_hbm.at[meta_v.at[0, 0]], rows[0], semg[0])
    pltpu.async_copy(m_hbm.at[meta_v.at[1, 0]], rows[1], semg[1])

    @pl.loop(0, NCH, step=NB)
    def _(j):
        def _chunk(jj, u):
            slot = lax.rem(jj, RB)
            pltpu.make_async_copy(m_hbm.at[meta_v.at[slot, 0]], rows[u],
                                  semg[u]).wait()
            _scale_rows(rows[u], meta_v.at[slot, 2])
            pltpu.async_copy(rows[u], agg_sh.at[meta_v.at[slot, 1]],
                             sems[u], add=True)

            @pl.when(jj + 2 < NCH)
            def _():
                nslot = lax.rem(jj + 2, RB)
                nu = (u + 2) % NB
                pltpu.make_async_copy(meta_hbm.at[w, jj + 2],
                                      meta_v.at[nslot],
                                      semm[(u + 2) % 3]).wait()

                @pl.when(jj >= 1)
                def _():
                    pltpu.make_async_copy(
                        rows[nu], agg_sh.at[meta_v.at[lax.rem(jj + 5, RB), 1]],
                        sems[nu]).wait()

                pltpu.async_copy(m_hbm.at[meta_v.at[nslot, 0]], rows[nu],
                                 semg[nu])

            @pl.when(jj + 4 < NCH)
            def _():
                pltpu.async_copy(meta_hbm.at[w, jj + 4],
                                 meta_v.at[lax.rem(jj + 4, RB)],
                                 semm[(u + 1) % 3])

        _chunk(j, 0)
        _chunk(j + 1, 1)
        _chunk(j + 2, 2)

    for jj in (NCH - 3, NCH - 2, NCH - 1):
        pltpu.make_async_copy(rows[jj % NB],
                              agg_sh.at[meta_v.at[jj % RB, 1]],
                              sems[jj % NB]).wait()

    plsc.subcore_barrier()
    pltpu.sync_copy(agg_sh.at[pl.ds(s * ROWS_PER_SUB, ROWS_PER_SUB)],
                    out_hbm.at[c, pl.ds(s * ROWS_PER_SUB, ROWS_PER_SUB)])


def _prep_edges(edge_index, edge_attr):
    src = edge_index[0].reshape(NW, EPW)
    dst = edge_index[1].reshape(NW, EPW)
    attr = lax.bitcast_convert_type(edge_attr, jnp.int32).reshape(NW, EPW)
    pad = EPWP - EPW
    src = jnp.pad(src, ((0, 0), (0, pad))).reshape(NW, NCH, C)
    dst = jnp.pad(dst, ((0, 0), (0, pad))).reshape(NW, NCH, C)
    attr = jnp.pad(attr, ((0, 0), (0, pad))).reshape(NW, NCH, C)
    return jnp.stack([src, dst, attr], axis=2)


def kernel(x, edge_index, edge_attr, weight, w_ih, w_hh, b_ih, b_hh):
    meta = _prep_edges(edge_index, edge_attr)
    zero = jnp.zeros((ROWS_PER_SUB, H), jnp.float32)
    w_ih_t = w_ih.T
    w_hh_t = w_hh.T
    b_ih2 = b_ih.reshape(1, 3 * H)
    b_hh2 = b_hh.reshape(1, 3 * H)
    h = x
    for i in range(L):
        m = _tc_matmul(h, weight[i])
        agg2 = _sc_agg_kernel()(m, meta, zero)
        h = _tc_gru(agg2, h, w_ih_t, w_hh_t, b_ih2, b_hh2)
    return h
